# baseline (speedup 1.0000x reference)
import functools

import jax
import jax.numpy as jnp
from jax.experimental import pallas as pl
from jax.experimental.pallas import tpu as pltpu

_REGIONS = [[0, 2], [1, 3], [4, 6, 8], [5, 7, 9], [10, 11, 12, 13],
            [14, 16, 18], [15, 17, 19], [20, 22, 24], [21, 23, 25],
            [26, 27, 28, 29, 30, 31]]
_F = 128
_N = 10
_H = 8
_C1 = 128
_C2 = 256


def _fwd(x_ref, w1_ref, b1_ref, m_ref, c1_ref, a1_ref, g1b_ref, w2g_ref,
         a2_ref, g2b_ref, cw1_ref, cb1_ref, cw2_ref, cb2_ref, o_ref):
    f32 = jnp.float32

    hw = []
    off = 0
    for i, ch in enumerate(_REGIONS):
        k = len(ch) * _F
        xi = x_ref[:, off:off + k]
        h = jnp.dot(xi, w1_ref[off:off + k, :], preferred_element_type=f32)
        h = jnp.maximum(h + b1_ref[i:i + 1, :], 0.0)
        off += k
        hwi = jnp.dot(h, m_ref[i * _F:(i + 1) * _F, :],
                      preferred_element_type=f32) + c1_ref[i:i + 1, :]
        hw.append(hwi)

    a_src, a_dst = [], []
    for i in range(_N):
        sd = jnp.dot(hw[i], a1_ref[...], preferred_element_type=f32)
        a_src.append(sd[:, 0:_H])
        a_dst.append(sd[:, _H:2 * _H])
    as_cat = jnp.concatenate(a_src, axis=1)

    hw2 = []
    for i in range(_N):
        ad_rep = jnp.concatenate([a_dst[i]] * _N, axis=1)
        e = ad_rep + as_cat
        e = jnp.where(e >= 0, e, 0.2 * e)
        m = functools.reduce(jnp.maximum,
                             [e[:, _H * j:_H * (j + 1)] for j in range(_N)])
        p = jnp.exp(e - jnp.concatenate([m] * _N, axis=1))
        den = functools.reduce(
            lambda a, b: a + b, [p[:, _H * j:_H * (j + 1)] for j in range(_N)])
        inv = 1.0 / den
        alpha = p * jnp.concatenate([inv] * _N, axis=1)
        _CH = 2
        w = _CH * _C1
        h2i = None
        for g in range(_H // _CH):
            acc = None
            for j in range(_N):
                a2l = alpha[:, _H * j + _CH * g:_H * j + _CH * (g + 1)]
                t = jnp.repeat(a2l, _C1, axis=1) \
                    * hw[j][:, g * w:(g + 1) * w]
                acc = t if acc is None else acc + t
            acc = acc + g1b_ref[:, g * w:(g + 1) * w]
            acc = jnp.where(acc > 0, acc,
                            jnp.exp(jnp.minimum(acc, 0.0)) - 1.0)
            part = jnp.dot(acc, w2g_ref[g * w:(g + 1) * w, :],
                           preferred_element_type=f32)
            h2i = part if h2i is None else h2i + part
        hw2.append(h2i)

    sd2 = [jnp.dot(hw2[i], a2_ref[...], preferred_element_type=f32)
           for i in range(_N)]
    ss = jnp.concatenate([s[:, 0:1] for s in sd2], axis=1)
    wacc = None
    for i in range(_N):
        e2 = sd2[i][:, 1:2] + ss
        e2 = jnp.where(e2 >= 0, e2, 0.2 * e2)
        p2 = jnp.exp(e2 - jnp.max(e2, axis=1, keepdims=True))
        a2v = p2 / jnp.sum(p2, axis=1, keepdims=True)
        wacc = a2v if wacc is None else wacc + a2v
    wmean = wacc * (1.0 / _N)
    pooled = None
    for j in range(_N):
        t = wmean[:, j:j + 1] * hw2[j]
        pooled = t if pooled is None else pooled + t
    pooled = pooled + g2b_ref[...]

    hc = jnp.dot(pooled, cw1_ref[...], preferred_element_type=f32)
    hc = jnp.maximum(hc + cb1_ref[...], 0.0)
    o_ref[...] = jnp.dot(hc, cw2_ref[...],
                         preferred_element_type=f32) + cb2_ref[...]


def kernel(x, enc_w1, enc_b1, enc_w2, enc_b2, wq, wk, gat1_w, gat1_att_src,
           gat1_att_dst, gat1_bias, gat2_w, gat2_att_src, gat2_att_dst,
           gat2_bias, cls_w1, cls_b1, cls_w2, cls_b2):
    b = x.shape[0]
    bb = min(128, b)
    perm = [c for ch in _REGIONS for c in ch]
    x2 = jnp.concatenate([x[:, c] for c in perm], axis=1)

    w1_all = jnp.concatenate([w.T for w in enc_w1], axis=0)
    b1_all = jnp.stack(enc_b1)
    m_all = jnp.concatenate([(gat1_w @ w2).T for w2 in enc_w2],
                            axis=0)
    c1_all = jnp.stack([b2 @ gat1_w.T for b2 in enc_b2])
    eye_h = jnp.eye(_H, dtype=jnp.float32)
    s_src = jnp.einsum('hc,hk->hck', gat1_att_src, eye_h).reshape(_H * _C1, _H)
    s_dst = jnp.einsum('hc,hk->hck', gat1_att_dst, eye_h).reshape(_H * _C1, _H)
    a1 = jnp.concatenate([s_src, s_dst], axis=1)
    g1b = gat1_bias.reshape(1, _H * _C1)
    w2g = gat2_w.T
    a2 = jnp.concatenate([gat2_att_src.T, gat2_att_dst.T], axis=1)
    g2b = gat2_bias.reshape(1, _C2)
    cw1 = cls_w1.T
    cb1 = cls_b1.reshape(1, -1)
    cw2 = cls_w2.T
    cb2 = cls_b2.reshape(1, -1)

    def full(a):
        return pl.BlockSpec(a.shape, lambda i: (0,) * a.ndim)

    weights = (w1_all, b1_all, m_all, c1_all, a1, g1b, w2g, a2, g2b,
               cw1, cb1, cw2, cb2)
    out = pl.pallas_call(
        _fwd,
        grid=(b // bb,),
        in_specs=[pl.BlockSpec((bb, 32 * _F), lambda i: (i, 0))]
                 + [full(w) for w in weights],
        out_specs=pl.BlockSpec((bb, 2), lambda i: (i, 0)),
        out_shape=jax.ShapeDtypeStruct((b, 2), jnp.float32),
        compiler_params=pltpu.CompilerParams(
            dimension_semantics=("parallel",),
            vmem_limit_bytes=128 * 1024 * 1024),
    )(x2, *weights)
    return out


# speedup vs baseline: 2.0891x; 2.0891x over previous
import functools

import jax
import jax.numpy as jnp
from jax.experimental import pallas as pl
from jax.experimental.pallas import tpu as pltpu

_REGIONS = [[0, 2], [1, 3], [4, 6, 8], [5, 7, 9], [10, 11, 12, 13],
            [14, 16, 18], [15, 17, 19], [20, 22, 24], [21, 23, 25],
            [26, 27, 28, 29, 30, 31]]
_F = 128
_N = 10
_H = 8
_C1 = 128
_C2 = 256


def _fwd(x_ref, w1_ref, b1_ref, m_ref, c1_ref, g1b_ref, w2g_ref,
         a2_ref, g2b_ref, cw1_ref, cb1_ref, cw2_ref, cb2_ref, o_ref):
    f32 = jnp.float32

    hw, a_src, a_dst = [], [], []
    off = 0
    for i, ch in enumerate(_REGIONS):
        k = len(ch) * _F
        xi = x_ref[:, off:off + k]
        h = jnp.dot(xi, w1_ref[off:off + k, :], preferred_element_type=f32)
        h = jnp.maximum(h + b1_ref[i:i + 1, :], 0.0)
        off += k
        hwe = jnp.dot(h, m_ref[i * _F:(i + 1) * _F, :],
                      preferred_element_type=f32) + c1_ref[i:i + 1, :]
        hw.append(hwe[:, :_H * _C1])
        a_src.append(hwe[:, _H * _C1:_H * _C1 + _H])
        a_dst.append(hwe[:, _H * _C1 + _H:_H * _C1 + 2 * _H])
    as_cat = jnp.concatenate(a_src, axis=1)

    hw2 = []
    for i in range(_N):
        ad_rep = jnp.concatenate([a_dst[i]] * _N, axis=1)
        e = ad_rep + as_cat
        e = jnp.where(e >= 0, e, 0.2 * e)
        m = functools.reduce(jnp.maximum,
                             [e[:, _H * j:_H * (j + 1)] for j in range(_N)])
        p = jnp.exp(e - jnp.concatenate([m] * _N, axis=1))
        den = functools.reduce(
            lambda a, b: a + b, [p[:, _H * j:_H * (j + 1)] for j in range(_N)])
        inv = 1.0 / den
        alpha = p * jnp.concatenate([inv] * _N, axis=1)
        u = None
        for j in range(_N):
            aj = jnp.repeat(alpha[:, _H * j:_H * (j + 1)], _C1, axis=1)
            t = aj * hw[j]
            u = t if u is None else u + t
        u = u + g1b_ref[...]
        u = jnp.where(u > 0, u, jnp.exp(jnp.minimum(u, 0.0)) - 1.0)
        hw2.append(jnp.dot(u, w2g_ref[...], preferred_element_type=f32))

    sd2 = [jnp.dot(hw2[i], a2_ref[...], preferred_element_type=f32)
           for i in range(_N)]
    ss = jnp.concatenate([s[:, 0:1] for s in sd2], axis=1)
    wacc = None
    for i in range(_N):
        e2 = sd2[i][:, 1:2] + ss
        e2 = jnp.where(e2 >= 0, e2, 0.2 * e2)
        p2 = jnp.exp(e2 - jnp.max(e2, axis=1, keepdims=True))
        a2v = p2 / jnp.sum(p2, axis=1, keepdims=True)
        wacc = a2v if wacc is None else wacc + a2v
    wmean = wacc * (1.0 / _N)
    pooled = None
    for j in range(_N):
        t = wmean[:, j:j + 1] * hw2[j]
        pooled = t if pooled is None else pooled + t
    pooled = pooled + g2b_ref[...]

    hc = jnp.dot(pooled, cw1_ref[...], preferred_element_type=f32)
    hc = jnp.maximum(hc + cb1_ref[...], 0.0)
    o_ref[...] = jnp.dot(hc, cw2_ref[...],
                         preferred_element_type=f32) + cb2_ref[...]


def kernel(x, enc_w1, enc_b1, enc_w2, enc_b2, wq, wk, gat1_w, gat1_att_src,
           gat1_att_dst, gat1_bias, gat2_w, gat2_att_src, gat2_att_dst,
           gat2_bias, cls_w1, cls_b1, cls_w2, cls_b2):
    b = x.shape[0]
    bb = min(128, b)
    perm = [c for ch in _REGIONS for c in ch]
    x2 = jnp.concatenate([x[:, c] for c in perm], axis=1)

    w1_all = jnp.concatenate([w.T for w in enc_w1], axis=0)
    b1_all = jnp.stack(enc_b1)
    eye_h = jnp.eye(_H, dtype=jnp.float32)
    s_src = jnp.einsum('hc,hk->hck', gat1_att_src, eye_h).reshape(_H * _C1, _H)
    s_dst = jnp.einsum('hc,hk->hck', gat1_att_dst, eye_h).reshape(_H * _C1, _H)
    a1 = jnp.concatenate([s_src, s_dst], axis=1)
    m_all = jnp.concatenate(
        [jnp.concatenate([mi, mi @ a1], axis=1)
         for mi in [(gat1_w @ w2).T for w2 in enc_w2]], axis=0)
    c1_all = jnp.stack(
        [jnp.concatenate([ci, ci @ a1])
         for ci in [b2 @ gat1_w.T for b2 in enc_b2]])
    g1b = gat1_bias.reshape(1, _H * _C1)
    w2g = gat2_w.T
    a2 = jnp.concatenate([gat2_att_src.T, gat2_att_dst.T], axis=1)
    g2b = gat2_bias.reshape(1, _C2)
    cw1 = cls_w1.T
    cb1 = cls_b1.reshape(1, -1)
    cw2 = cls_w2.T
    cb2 = cls_b2.reshape(1, -1)

    def full(a):
        return pl.BlockSpec(a.shape, lambda i: (0,) * a.ndim)

    weights = (w1_all, b1_all, m_all, c1_all, g1b, w2g, a2, g2b,
               cw1, cb1, cw2, cb2)
    out = pl.pallas_call(
        _fwd,
        grid=(b // bb,),
        in_specs=[pl.BlockSpec((bb, 32 * _F), lambda i: (i, 0))]
                 + [full(w) for w in weights],
        out_specs=pl.BlockSpec((bb, 2), lambda i: (i, 0)),
        out_shape=jax.ShapeDtypeStruct((b, 2), jnp.float32),
        compiler_params=pltpu.CompilerParams(
            dimension_semantics=("parallel",),
            vmem_limit_bytes=128 * 1024 * 1024),
    )(x2, *weights)
    return out


# speedup vs baseline: 2.1101x; 1.0101x over previous
import functools

import jax
import jax.numpy as jnp
from jax.experimental import pallas as pl
from jax.experimental.pallas import tpu as pltpu

_REGIONS = [[0, 2], [1, 3], [4, 6, 8], [5, 7, 9], [10, 11, 12, 13],
            [14, 16, 18], [15, 17, 19], [20, 22, 24], [21, 23, 25],
            [26, 27, 28, 29, 30, 31]]
_F = 128
_N = 10
_H = 8
_C1 = 128
_C2 = 256


def _fwd(x_ref, w1_ref, b1_ref, m_ref, c1_ref, a1s_ref, c1s_ref, g1b_ref,
         w2g_ref, a2_ref, g2b_ref, cw1_ref, cb1_ref, cw2_ref, cb2_ref, o_ref):
    f32 = jnp.float32

    hw, a_src, a_dst = [], [], []
    off = 0
    for i, ch in enumerate(_REGIONS):
        k = len(ch) * _F
        xi = x_ref[:, off:off + k]
        h = jnp.dot(xi, w1_ref[off:off + k, :], preferred_element_type=f32)
        h = jnp.maximum(h + b1_ref[i:i + 1, :], 0.0)
        off += k
        hwi = jnp.dot(h, m_ref[i * _F:(i + 1) * _F, :],
                      preferred_element_type=f32) + c1_ref[i:i + 1, :]
        hw.append(hwi)
        sd = jnp.dot(h, a1s_ref[i * _F:(i + 1) * _F, :],
                     preferred_element_type=f32) + c1s_ref[i:i + 1, :]
        a_src.append(sd[:, 0:_H])
        a_dst.append(sd[:, _H:2 * _H])
    as_cat = jnp.concatenate(a_src, axis=1)

    hw2 = []
    for i in range(_N):
        ad_rep = jnp.concatenate([a_dst[i]] * _N, axis=1)
        e = ad_rep + as_cat
        e = jnp.where(e >= 0, e, 0.2 * e)
        m = functools.reduce(jnp.maximum,
                             [e[:, _H * j:_H * (j + 1)] for j in range(_N)])
        p = jnp.exp(e - jnp.concatenate([m] * _N, axis=1))
        den = functools.reduce(
            lambda a, b: a + b, [p[:, _H * j:_H * (j + 1)] for j in range(_N)])
        inv = 1.0 / den
        alpha = p * jnp.concatenate([inv] * _N, axis=1)
        u = None
        for j in range(_N):
            aj = jnp.repeat(alpha[:, _H * j:_H * (j + 1)], _C1, axis=1)
            t = aj * hw[j]
            u = t if u is None else u + t
        u = u + g1b_ref[...]
        u = jnp.where(u > 0, u, jnp.exp(jnp.minimum(u, 0.0)) - 1.0)
        hw2.append(jnp.dot(u, w2g_ref[...], preferred_element_type=f32))

    sd2 = [jnp.dot(hw2[i], a2_ref[...], preferred_element_type=f32)
           for i in range(_N)]
    ss = jnp.concatenate([s[:, 0:1] for s in sd2], axis=1)
    wacc = None
    for i in range(_N):
        e2 = sd2[i][:, 1:2] + ss
        e2 = jnp.where(e2 >= 0, e2, 0.2 * e2)
        p2 = jnp.exp(e2 - jnp.max(e2, axis=1, keepdims=True))
        a2v = p2 / jnp.sum(p2, axis=1, keepdims=True)
        wacc = a2v if wacc is None else wacc + a2v
    wmean = wacc * (1.0 / _N)
    pooled = None
    for j in range(_N):
        t = wmean[:, j:j + 1] * hw2[j]
        pooled = t if pooled is None else pooled + t
    pooled = pooled + g2b_ref[...]

    hc = jnp.dot(pooled, cw1_ref[...], preferred_element_type=f32)
    hc = jnp.maximum(hc + cb1_ref[...], 0.0)
    o_ref[...] = jnp.dot(hc, cw2_ref[...],
                         preferred_element_type=f32) + cb2_ref[...]


def kernel(x, enc_w1, enc_b1, enc_w2, enc_b2, wq, wk, gat1_w, gat1_att_src,
           gat1_att_dst, gat1_bias, gat2_w, gat2_att_src, gat2_att_dst,
           gat2_bias, cls_w1, cls_b1, cls_w2, cls_b2):
    b = x.shape[0]
    bb = min(128, b)
    perm = [c for ch in _REGIONS for c in ch]
    x2 = jnp.concatenate([x[:, c] for c in perm], axis=1)

    w1_all = jnp.concatenate([w.T for w in enc_w1], axis=0)
    b1_all = jnp.stack(enc_b1)
    eye_h = jnp.eye(_H, dtype=jnp.float32)
    s_src = jnp.einsum('hc,hk->hck', gat1_att_src, eye_h).reshape(_H * _C1, _H)
    s_dst = jnp.einsum('hc,hk->hck', gat1_att_dst, eye_h).reshape(_H * _C1, _H)
    a1 = jnp.concatenate([s_src, s_dst], axis=1)
    m_list = [(gat1_w @ w2).T for w2 in enc_w2]
    c1_list = [b2 @ gat1_w.T for b2 in enc_b2]
    m_all = jnp.concatenate(m_list, axis=0)
    c1_all = jnp.stack(c1_list)
    a1s_all = jnp.concatenate([mi @ a1 for mi in m_list], axis=0)
    c1s_all = jnp.stack([ci @ a1 for ci in c1_list])
    g1b = gat1_bias.reshape(1, _H * _C1)
    w2g = gat2_w.T
    a2 = jnp.concatenate([gat2_att_src.T, gat2_att_dst.T], axis=1)
    g2b = gat2_bias.reshape(1, _C2)
    cw1 = cls_w1.T
    cb1 = cls_b1.reshape(1, -1)
    cw2 = cls_w2.T
    cb2 = cls_b2.reshape(1, -1)

    def full(a):
        return pl.BlockSpec(a.shape, lambda i: (0,) * a.ndim)

    weights = (w1_all, b1_all, m_all, c1_all, a1s_all, c1s_all, g1b, w2g,
               a2, g2b, cw1, cb1, cw2, cb2)
    out = pl.pallas_call(
        _fwd,
        grid=(b // bb,),
        in_specs=[pl.BlockSpec((bb, 32 * _F), lambda i: (i, 0))]
                 + [full(w) for w in weights],
        out_specs=pl.BlockSpec((bb, 2), lambda i: (i, 0)),
        out_shape=jax.ShapeDtypeStruct((b, 2), jnp.float32),
        compiler_params=pltpu.CompilerParams(
            dimension_semantics=("parallel",),
            vmem_limit_bytes=128 * 1024 * 1024),
    )(x2, *weights)
    return out


# speedup vs baseline: 2.6634x; 1.2622x over previous
import functools

import jax
import jax.numpy as jnp
from jax.experimental import pallas as pl
from jax.experimental.pallas import tpu as pltpu

_REGIONS = [[0, 2], [1, 3], [4, 6, 8], [5, 7, 9], [10, 11, 12, 13],
            [14, 16, 18], [15, 17, 19], [20, 22, 24], [21, 23, 25],
            [26, 27, 28, 29, 30, 31]]
_F = 128
_N = 10
_H = 8
_C1 = 128
_C2 = 256


def _fwd(x_ref, w1_ref, b1_ref, m_ref, c1_ref, a1s_ref, c1s_ref, g1b_ref,
         w2g_ref, a2_ref, g2b_ref, cw1_ref, cb1_ref, cw2_ref, cb2_ref, o_ref):
    f32 = jnp.float32

    hw, a_src, a_dst = [], [], []
    off = 0
    for i, ch in enumerate(_REGIONS):
        k = len(ch) * _F
        xi = x_ref[:, off:off + k]
        h = jnp.dot(xi, w1_ref[off:off + k, :], preferred_element_type=f32)
        h = jnp.maximum(h + b1_ref[i:i + 1, :], 0.0)
        off += k
        hwi = jnp.dot(h, m_ref[i * _F:(i + 1) * _F, :],
                      preferred_element_type=f32) + c1_ref[i:i + 1, :]
        hw.append(hwi)
        sd = jnp.dot(h, a1s_ref[i * _F:(i + 1) * _F, :],
                     preferred_element_type=f32) + c1s_ref[i:i + 1, :]
        a_src.append(sd[:, 0:_H])
        a_dst.append(sd[:, _H:2 * _H])
    as_cat = jnp.concatenate(a_src, axis=1)

    _W = _N * _H
    ad_cat = jnp.concatenate(a_dst, axis=1)
    e = jnp.concatenate(
        [ad_cat + jnp.concatenate([as_cat[:, _H * j:_H * (j + 1)]] * _N,
                                  axis=1) for j in range(_N)], axis=1)
    e = jnp.where(e >= 0, e, 0.2 * e)
    m = functools.reduce(jnp.maximum,
                         [e[:, _W * j:_W * (j + 1)] for j in range(_N)])
    p = jnp.exp(e - jnp.concatenate([m] * _N, axis=1))
    den = functools.reduce(
        lambda a, b: a + b, [p[:, _W * j:_W * (j + 1)] for j in range(_N)])
    inv = 1.0 / den
    alphaf = p * jnp.concatenate([inv] * _N, axis=1)

    hw2 = []
    for i in range(_N):
        u = None
        for j in range(_N):
            aj = jnp.repeat(
                alphaf[:, _W * j + _H * i:_W * j + _H * (i + 1)], _C1, axis=1)
            t = aj * hw[j]
            u = t if u is None else u + t
        u = u + g1b_ref[...]
        u = jnp.where(u > 0, u, jnp.exp(jnp.minimum(u, 0.0)) - 1.0)
        hw2.append(jnp.dot(u, w2g_ref[...], preferred_element_type=f32))

    sd2 = [jnp.dot(hw2[i], a2_ref[...], preferred_element_type=f32)
           for i in range(_N)]
    ss = jnp.concatenate([s[:, 0:1] for s in sd2], axis=1)
    wacc = None
    for i in range(_N):
        e2 = sd2[i][:, 1:2] + ss
        e2 = jnp.where(e2 >= 0, e2, 0.2 * e2)
        p2 = jnp.exp(e2 - jnp.max(e2, axis=1, keepdims=True))
        a2v = p2 / jnp.sum(p2, axis=1, keepdims=True)
        wacc = a2v if wacc is None else wacc + a2v
    wmean = wacc * (1.0 / _N)
    pooled = None
    for j in range(_N):
        t = wmean[:, j:j + 1] * hw2[j]
        pooled = t if pooled is None else pooled + t
    pooled = pooled + g2b_ref[...]

    hc = jnp.dot(pooled, cw1_ref[...], preferred_element_type=f32)
    hc = jnp.maximum(hc + cb1_ref[...], 0.0)
    o_ref[...] = jnp.dot(hc, cw2_ref[...],
                         preferred_element_type=f32) + cb2_ref[...]


def kernel(x, enc_w1, enc_b1, enc_w2, enc_b2, wq, wk, gat1_w, gat1_att_src,
           gat1_att_dst, gat1_bias, gat2_w, gat2_att_src, gat2_att_dst,
           gat2_bias, cls_w1, cls_b1, cls_w2, cls_b2):
    b = x.shape[0]
    bb = min(128, b)
    perm = [c for ch in _REGIONS for c in ch]
    x2 = jnp.concatenate([x[:, c] for c in perm], axis=1)

    w1_all = jnp.concatenate([w.T for w in enc_w1], axis=0)
    b1_all = jnp.stack(enc_b1)
    eye_h = jnp.eye(_H, dtype=jnp.float32)
    s_src = jnp.einsum('hc,hk->hck', gat1_att_src, eye_h).reshape(_H * _C1, _H)
    s_dst = jnp.einsum('hc,hk->hck', gat1_att_dst, eye_h).reshape(_H * _C1, _H)
    a1 = jnp.concatenate([s_src, s_dst], axis=1)
    m_list = [(gat1_w @ w2).T for w2 in enc_w2]
    c1_list = [b2 @ gat1_w.T for b2 in enc_b2]
    m_all = jnp.concatenate(m_list, axis=0)
    c1_all = jnp.stack(c1_list)
    a1s_all = jnp.concatenate([mi @ a1 for mi in m_list], axis=0)
    c1s_all = jnp.stack([ci @ a1 for ci in c1_list])
    g1b = gat1_bias.reshape(1, _H * _C1)
    w2g = gat2_w.T
    a2 = jnp.concatenate([gat2_att_src.T, gat2_att_dst.T], axis=1)
    g2b = gat2_bias.reshape(1, _C2)
    cw1 = cls_w1.T
    cb1 = cls_b1.reshape(1, -1)
    cw2 = cls_w2.T
    cb2 = cls_b2.reshape(1, -1)

    def full(a):
        return pl.BlockSpec(a.shape, lambda i: (0,) * a.ndim)

    weights = (w1_all, b1_all, m_all, c1_all, a1s_all, c1s_all, g1b, w2g,
               a2, g2b, cw1, cb1, cw2, cb2)
    out = pl.pallas_call(
        _fwd,
        grid=(b // bb,),
        in_specs=[pl.BlockSpec((bb, 32 * _F), lambda i: (i, 0))]
                 + [full(w) for w in weights],
        out_specs=pl.BlockSpec((bb, 2), lambda i: (i, 0)),
        out_shape=jax.ShapeDtypeStruct((b, 2), jnp.float32),
        compiler_params=pltpu.CompilerParams(
            dimension_semantics=("parallel",),
            vmem_limit_bytes=128 * 1024 * 1024),
    )(x2, *weights)
    return out
